# baseline (speedup 1.0000x reference)
import functools

import jax
import jax.numpy as jnp
from jax import lax
from jax.experimental import pallas as pl
from jax.experimental.pallas import tpu as pltpu
from jax.experimental.pallas import tpu_sc as plsc

N = 10000
E = 160000
D = 256

NPAD = 10240
RPT = NPAD // 16
ECHUNK = 64
NSLAB = 16
NJ = 160
EPAD = NSLAB * NJ * ECHUNK
NJH = NJ // 2
NSTAGE = 4
NJC = NJ // NSTAGE
NBUF = 4
FH = 128

_mesh = plsc.VectorSubcoreMesh(core_axis_name="c", subcore_axis_name="s")


@functools.partial(
    pl.kernel,
    out_type=jax.ShapeDtypeStruct((2, NPAD), jnp.float32),
    mesh=_mesh,
    scratch_types=[
        pltpu.VMEM((NJH, ECHUNK), jnp.int32),
        pltpu.VMEM((NJH, ECHUNK), jnp.float32),
        pltpu.VMEM_SHARED((NPAD,), jnp.float32),
    ],
)
def _deg_kernel(col_ref, w_ref, z_ref, deg_ref, col_v, w_v, acc_sh):
    cid = lax.axis_index("c")
    sid = lax.axis_index("s")
    pltpu.sync_copy(col_ref.at[sid, pl.ds(cid * NJH, NJH)], col_v)
    pltpu.sync_copy(w_ref.at[sid, pl.ds(cid * NJH, NJH)], w_v)
    pltpu.sync_copy(z_ref.at[pl.ds(sid * RPT, RPT)],
                    acc_sh.at[pl.ds(sid * RPT, RPT)])
    plsc.subcore_barrier()

    @pl.loop(0, NJH)
    def _(j):
        pltpu.sync_copy(w_v.at[j], acc_sh.at[col_v.at[j]], add=True)

    plsc.subcore_barrier()
    pltpu.sync_copy(acc_sh.at[pl.ds(sid * RPT, RPT)],
                    deg_ref.at[cid, pl.ds(sid * RPT, RPT)])


@functools.partial(
    pl.kernel,
    out_type=jax.ShapeDtypeStruct((2, NPAD, FH), jnp.float32),
    mesh=_mesh,
    scratch_types=[
        pltpu.VMEM((NJC, ECHUNK), jnp.int32),
        pltpu.VMEM((NJC, ECHUNK), jnp.int32),
        pltpu.VMEM((NJC, ECHUNK), jnp.float32),
        [pltpu.VMEM((ECHUNK, FH), jnp.float32)] * NBUF,
        pltpu.VMEM_SHARED((NPAD, FH), jnp.float32),
        [pltpu.SemaphoreType.DMA] * NBUF,
        [pltpu.SemaphoreType.DMA] * NBUF,
    ],
)
def _prop_kernel(h_ref, row_ref, col_ref, w_ref, agg_ref,
                 row_v, col_v, w_v, bufs, acc_sh, gsems, ssems):
    cid = lax.axis_index("c")
    sid = lax.axis_index("s")
    pltpu.sync_copy(h_ref.at[cid, pl.ds(sid * RPT, RPT)],
                    acc_sh.at[pl.ds(sid * RPT, RPT)])
    plsc.subcore_barrier()

    src = h_ref.at[cid]

    def _step(j, b, g):
        gbuf = bufs[b]
        pltpu.make_async_copy(src.at[row_v.at[j]], gbuf, gsems[b]).wait()

        @plsc.parallel_loop(0, ECHUNK // 16, 1, unroll=2)
        def _(grp):
            wvec = w_v[j, pl.ds(grp * 16, 16)]
            for l in range(16):
                wv = jnp.full((16,), wvec[l], jnp.float32)
                e = grp * 16 + l
                for k in range(FH // 16):
                    sl = pl.ds(k * 16, 16)
                    gbuf[e, sl] = gbuf[e, sl] * wv

        pltpu.async_copy(gbuf, acc_sh.at[col_v.at[j]], ssems[b], add=True)
        bn = (b + 2) % NBUF
        jn = j + 2

        @pl.when(jn < NJC)
        def _():
            def _refill():
                pltpu.make_async_copy(
                    bufs[bn], acc_sh.at[col_v.at[j]], ssems[bn]).wait()
                pltpu.async_copy(src.at[row_v.at[jn]], bufs[bn], gsems[bn])

            if b >= 2:
                _refill()
            else:
                @pl.when(g >= 1)
                def _():
                    _refill()

                @pl.when(g < 1)
                def _():
                    pltpu.async_copy(src.at[row_v.at[jn]],
                                     bufs[bn], gsems[bn])

    for stage in range(NSTAGE):
        pltpu.sync_copy(row_ref.at[sid, pl.ds(stage * NJC, NJC)], row_v)
        pltpu.sync_copy(col_ref.at[sid, pl.ds(stage * NJC, NJC)], col_v)
        pltpu.sync_copy(w_ref.at[sid, pl.ds(stage * NJC, NJC)], w_v)
        pltpu.async_copy(src.at[row_v.at[0]], bufs[0], gsems[0])
        pltpu.async_copy(src.at[row_v.at[1]], bufs[1], gsems[1])

        @pl.loop(0, NJC // NBUF)
        def _(g):
            for b in range(NBUF):
                _step(g * NBUF + b, b, g)

        for b in range(NBUF):
            pltpu.make_async_copy(
                bufs[b], acc_sh.at[col_v.at[0]], ssems[b]).wait()

    plsc.subcore_barrier()
    pltpu.sync_copy(acc_sh.at[pl.ds(sid * RPT, RPT)],
                    agg_ref.at[cid, pl.ds(sid * RPT, RPT)])


def _dinv_of(deg_ref):
    return lax.rsqrt(1.0 + deg_ref[0] + deg_ref[1])


def _mm1_body(x_ref, w_ref, deg_ref, o_ref, dinv_ref):
    dinv = _dinv_of(deg_ref)
    dinv_ref[...] = dinv
    h = jnp.dot(x_ref[...], w_ref[...], precision=lax.Precision.HIGHEST)
    h = h * dinv
    o_ref[0] = h[:, :FH]
    o_ref[1] = h[:, FH:]


def _mm1_call(x_pad, W1, deg3):
    return pl.pallas_call(
        _mm1_body,
        grid=(NPAD // 256,),
        in_specs=[
            pl.BlockSpec((256, D), lambda m: (m, 0)),
            pl.BlockSpec((D, D), lambda m: (0, 0)),
            pl.BlockSpec((2, 256, 1), lambda m: (0, m, 0)),
        ],
        out_specs=[pl.BlockSpec((2, 256, FH), lambda m: (0, m, 0)),
                   pl.BlockSpec((256, 1), lambda m: (m, 0))],
        out_shape=[jax.ShapeDtypeStruct((2, NPAD, FH), jnp.float32),
                   jax.ShapeDtypeStruct((NPAD, 1), jnp.float32)],
    )(x_pad, W1, deg3)


def _mid_body(agg_ref, dinv_in_ref, b1_ref, w2_ref, o_ref):
    dinv = dinv_in_ref[...]
    u = jnp.concatenate([agg_ref[0], agg_ref[1]], axis=1)
    out1 = jax.nn.relu(u * dinv + b1_ref[...])
    h2 = jnp.dot(out1, w2_ref[...], precision=lax.Precision.HIGHEST)
    h2 = h2 * dinv
    o_ref[0] = h2[:, :FH]
    o_ref[1] = h2[:, FH:]


def _mid_call(agg1, dinv_col, b1, W2):
    return pl.pallas_call(
        _mid_body,
        grid=(NPAD // 256,),
        in_specs=[
            pl.BlockSpec((2, 256, FH), lambda m: (0, m, 0)),
            pl.BlockSpec((256, 1), lambda m: (m, 0)),
            pl.BlockSpec((1, D), lambda m: (0, 0)),
            pl.BlockSpec((D, D), lambda m: (0, 0)),
        ],
        out_specs=pl.BlockSpec((2, 256, FH), lambda m: (0, m, 0)),
        out_shape=jax.ShapeDtypeStruct((2, NPAD, FH), jnp.float32),
    )(agg1, dinv_col, b1, W2)


def _fin_body(agg_ref, dinv_in_ref, b2_ref, o_ref):
    u = jnp.concatenate([agg_ref[0], agg_ref[1]], axis=1)
    o_ref[...] = u * dinv_in_ref[...] + b2_ref[...]


def _fin_call(agg2, dinv_col, b2):
    return pl.pallas_call(
        _fin_body,
        grid=(NPAD // 256,),
        in_specs=[
            pl.BlockSpec((2, 256, FH), lambda m: (0, m, 0)),
            pl.BlockSpec((256, 1), lambda m: (m, 0)),
            pl.BlockSpec((1, D), lambda m: (0, 0)),
        ],
        out_specs=pl.BlockSpec((256, D), lambda m: (m, 0)),
        out_shape=jax.ShapeDtypeStruct((NPAD, D), jnp.float32),
    )(agg2, dinv_col, b2)


def kernel(x, edge_index, edge_weight, W1, b1, W2, b2):
    row = edge_index[0].astype(jnp.int32)
    col = edge_index[1].astype(jnp.int32)
    w = edge_weight.astype(jnp.float32)
    pad = EPAD - E
    padidx = N + (jnp.arange(pad, dtype=jnp.int32) % (NPAD - N))
    row_t = jnp.concatenate([row, padidx]).reshape(NSLAB, NJ, ECHUNK)
    col_t = jnp.concatenate([col, padidx]).reshape(NSLAB, NJ, ECHUNK)
    w_t = jnp.concatenate([w, jnp.zeros((pad,), jnp.float32)]
                          ).reshape(NSLAB, NJ, ECHUNK)
    x_pad = jnp.pad(x, ((0, NPAD - N), (0, 0)))
    b1r = b1.reshape(1, D)
    b2r = b2.reshape(1, D)

    zcol = jnp.zeros((NPAD,), jnp.float32)
    deg3 = _deg_kernel(col_t, w_t, zcol).reshape(2, NPAD, 1)
    h1, dinv_col = _mm1_call(x_pad, W1, deg3)
    agg1 = _prop_kernel(h1, row_t, col_t, w_t)
    h2 = _mid_call(agg1, dinv_col, b1r, W2)
    agg2 = _prop_kernel(h2, row_t, col_t, w_t)
    out = _fin_call(agg2, dinv_col, b2r)
    return out[:N]


# speedup vs baseline: 1.0397x; 1.0397x over previous
import functools

import jax
import jax.numpy as jnp
from jax import lax
from jax.experimental import pallas as pl
from jax.experimental.pallas import tpu as pltpu
from jax.experimental.pallas import tpu_sc as plsc

N = 10000
E = 160000
D = 256

NPAD = 10240
RPT = NPAD // 16
ECHUNK = 128
NSLAB = 16
NJ = 80
EPAD = NSLAB * NJ * ECHUNK
NJH = NJ // 2
NSTAGE = 2
NJC = NJ // NSTAGE
NBUF = 2
FH = 128

_mesh = plsc.VectorSubcoreMesh(core_axis_name="c", subcore_axis_name="s")


@functools.partial(
    pl.kernel,
    out_type=jax.ShapeDtypeStruct((2, NPAD), jnp.float32),
    mesh=_mesh,
    scratch_types=[
        pltpu.VMEM((NJH, ECHUNK), jnp.int32),
        pltpu.VMEM((NJH, ECHUNK), jnp.float32),
        pltpu.VMEM_SHARED((NPAD,), jnp.float32),
    ],
)
def _deg_kernel(col_ref, w_ref, z_ref, deg_ref, col_v, w_v, acc_sh):
    cid = lax.axis_index("c")
    sid = lax.axis_index("s")
    pltpu.sync_copy(col_ref.at[sid, pl.ds(cid * NJH, NJH)], col_v)
    pltpu.sync_copy(w_ref.at[sid, pl.ds(cid * NJH, NJH)], w_v)
    pltpu.sync_copy(z_ref.at[pl.ds(sid * RPT, RPT)],
                    acc_sh.at[pl.ds(sid * RPT, RPT)])
    plsc.subcore_barrier()

    @pl.loop(0, NJH)
    def _(j):
        pltpu.sync_copy(w_v.at[j], acc_sh.at[col_v.at[j]], add=True)

    plsc.subcore_barrier()
    pltpu.sync_copy(acc_sh.at[pl.ds(sid * RPT, RPT)],
                    deg_ref.at[cid, pl.ds(sid * RPT, RPT)])


@functools.partial(
    pl.kernel,
    out_type=jax.ShapeDtypeStruct((2, NPAD, FH), jnp.float32),
    mesh=_mesh,
    scratch_types=[
        pltpu.VMEM((NJC, ECHUNK), jnp.int32),
        pltpu.VMEM((NJC, ECHUNK), jnp.int32),
        pltpu.VMEM((NJC, ECHUNK), jnp.float32),
        [pltpu.VMEM((ECHUNK, FH), jnp.float32)] * NBUF,
        pltpu.VMEM_SHARED((NPAD, FH), jnp.float32),
        [pltpu.SemaphoreType.DMA] * NBUF,
        [pltpu.SemaphoreType.DMA] * NBUF,
    ],
)
def _prop_kernel(h_ref, row_ref, col_ref, w_ref, agg_ref,
                 row_v, col_v, w_v, bufs, acc_sh, gsems, ssems):
    cid = lax.axis_index("c")
    sid = lax.axis_index("s")
    pltpu.sync_copy(h_ref.at[cid, pl.ds(sid * RPT, RPT)],
                    acc_sh.at[pl.ds(sid * RPT, RPT)])
    plsc.subcore_barrier()

    src = h_ref.at[cid]

    def _step(j, b, g):
        gbuf = bufs[b]
        bn = 1 - b
        pltpu.make_async_copy(src.at[row_v.at[j]], gbuf, gsems[b]).wait()
        jn = j + 1

        @pl.when(jn < NJC)
        def _():
            def _refill():
                pltpu.make_async_copy(
                    bufs[bn], acc_sh.at[col_v.at[j]], ssems[bn]).wait()
                pltpu.async_copy(src.at[row_v.at[jn]], bufs[bn], gsems[bn])

            if b == 1:
                _refill()
            else:
                @pl.when(g >= 1)
                def _():
                    _refill()

                @pl.when(g < 1)
                def _():
                    pltpu.async_copy(src.at[row_v.at[jn]],
                                     bufs[bn], gsems[bn])

        @plsc.parallel_loop(0, ECHUNK // 16, 1, unroll=2)
        def _(grp):
            wvec = w_v[j, pl.ds(grp * 16, 16)]
            for l in range(16):
                wv = jnp.full((16,), wvec[l], jnp.float32)
                e = grp * 16 + l
                for k in range(FH // 16):
                    sl = pl.ds(k * 16, 16)
                    gbuf[e, sl] = gbuf[e, sl] * wv

        pltpu.async_copy(gbuf, acc_sh.at[col_v.at[j]], ssems[b], add=True)

    for stage in range(NSTAGE):
        pltpu.sync_copy(row_ref.at[sid, pl.ds(stage * NJC, NJC)], row_v)
        pltpu.sync_copy(col_ref.at[sid, pl.ds(stage * NJC, NJC)], col_v)
        pltpu.sync_copy(w_ref.at[sid, pl.ds(stage * NJC, NJC)], w_v)
        pltpu.async_copy(src.at[row_v.at[0]], bufs[0], gsems[0])

        @pl.loop(0, NJC // NBUF)
        def _(g):
            for b in range(NBUF):
                _step(g * NBUF + b, b, g)

        pltpu.make_async_copy(
            bufs[(NJC - 1) % NBUF], acc_sh.at[col_v.at[0]],
            ssems[(NJC - 1) % NBUF]).wait()

    plsc.subcore_barrier()
    pltpu.sync_copy(acc_sh.at[pl.ds(sid * RPT, RPT)],
                    agg_ref.at[cid, pl.ds(sid * RPT, RPT)])


def _dinv_of(deg_ref):
    return lax.rsqrt(1.0 + deg_ref[0] + deg_ref[1])


def _mm1_body(x_ref, w_ref, deg_ref, o_ref, dinv_ref):
    dinv = _dinv_of(deg_ref)
    dinv_ref[...] = dinv
    h = jnp.dot(x_ref[...], w_ref[...], precision=lax.Precision.HIGHEST)
    h = h * dinv
    o_ref[0] = h[:, :FH]
    o_ref[1] = h[:, FH:]


def _mm1_call(x_pad, W1, deg3):
    return pl.pallas_call(
        _mm1_body,
        grid=(NPAD // 256,),
        in_specs=[
            pl.BlockSpec((256, D), lambda m: (m, 0)),
            pl.BlockSpec((D, D), lambda m: (0, 0)),
            pl.BlockSpec((2, 256, 1), lambda m: (0, m, 0)),
        ],
        out_specs=[pl.BlockSpec((2, 256, FH), lambda m: (0, m, 0)),
                   pl.BlockSpec((256, 1), lambda m: (m, 0))],
        out_shape=[jax.ShapeDtypeStruct((2, NPAD, FH), jnp.float32),
                   jax.ShapeDtypeStruct((NPAD, 1), jnp.float32)],
    )(x_pad, W1, deg3)


def _mid_body(agg_ref, dinv_in_ref, b1_ref, w2_ref, o_ref):
    dinv = dinv_in_ref[...]
    u = jnp.concatenate([agg_ref[0], agg_ref[1]], axis=1)
    out1 = jax.nn.relu(u * dinv + b1_ref[...])
    h2 = jnp.dot(out1, w2_ref[...], precision=lax.Precision.HIGHEST)
    h2 = h2 * dinv
    o_ref[0] = h2[:, :FH]
    o_ref[1] = h2[:, FH:]


def _mid_call(agg1, dinv_col, b1, W2):
    return pl.pallas_call(
        _mid_body,
        grid=(NPAD // 256,),
        in_specs=[
            pl.BlockSpec((2, 256, FH), lambda m: (0, m, 0)),
            pl.BlockSpec((256, 1), lambda m: (m, 0)),
            pl.BlockSpec((1, D), lambda m: (0, 0)),
            pl.BlockSpec((D, D), lambda m: (0, 0)),
        ],
        out_specs=pl.BlockSpec((2, 256, FH), lambda m: (0, m, 0)),
        out_shape=jax.ShapeDtypeStruct((2, NPAD, FH), jnp.float32),
    )(agg1, dinv_col, b1, W2)


def _fin_body(agg_ref, dinv_in_ref, b2_ref, o_ref):
    u = jnp.concatenate([agg_ref[0], agg_ref[1]], axis=1)
    o_ref[...] = u * dinv_in_ref[...] + b2_ref[...]


def _fin_call(agg2, dinv_col, b2):
    return pl.pallas_call(
        _fin_body,
        grid=(NPAD // 256,),
        in_specs=[
            pl.BlockSpec((2, 256, FH), lambda m: (0, m, 0)),
            pl.BlockSpec((256, 1), lambda m: (m, 0)),
            pl.BlockSpec((1, D), lambda m: (0, 0)),
        ],
        out_specs=pl.BlockSpec((256, D), lambda m: (m, 0)),
        out_shape=jax.ShapeDtypeStruct((NPAD, D), jnp.float32),
    )(agg2, dinv_col, b2)


def kernel(x, edge_index, edge_weight, W1, b1, W2, b2):
    row = edge_index[0].astype(jnp.int32)
    col = edge_index[1].astype(jnp.int32)
    w = edge_weight.astype(jnp.float32)
    pad = EPAD - E
    padidx = N + (jnp.arange(pad, dtype=jnp.int32) % (NPAD - N))
    row_t = jnp.concatenate([row, padidx]).reshape(NSLAB, NJ, ECHUNK)
    col_t = jnp.concatenate([col, padidx]).reshape(NSLAB, NJ, ECHUNK)
    w_t = jnp.concatenate([w, jnp.zeros((pad,), jnp.float32)]
                          ).reshape(NSLAB, NJ, ECHUNK)
    x_pad = jnp.pad(x, ((0, NPAD - N), (0, 0)))
    b1r = b1.reshape(1, D)
    b2r = b2.reshape(1, D)

    zcol = jnp.zeros((NPAD,), jnp.float32)
    deg3 = _deg_kernel(col_t, w_t, zcol).reshape(2, NPAD, 1)
    h1, dinv_col = _mm1_call(x_pad, W1, deg3)
    agg1 = _prop_kernel(h1, row_t, col_t, w_t)
    h2 = _mid_call(agg1, dinv_col, b1r, W2)
    agg2 = _prop_kernel(h2, row_t, col_t, w_t)
    out = _fin_call(agg2, dinv_col, b2r)
    return out[:N]


# speedup vs baseline: 1.0565x; 1.0161x over previous
import functools

import jax
import jax.numpy as jnp
from jax import lax
from jax.experimental import pallas as pl
from jax.experimental.pallas import tpu as pltpu
from jax.experimental.pallas import tpu_sc as plsc

N = 10000
E = 160000
D = 256

NPAD = 10240
RPT = NPAD // 16
ECHUNK = 128
NSLAB = 16
NJ = 80
EPAD = NSLAB * NJ * ECHUNK
NJH = NJ // 2
FH = 128

_mesh = plsc.VectorSubcoreMesh(core_axis_name="c", subcore_axis_name="s")


@functools.partial(
    pl.kernel,
    out_type=jax.ShapeDtypeStruct((2, NPAD), jnp.float32),
    mesh=_mesh,
    scratch_types=[
        pltpu.VMEM((NJH, ECHUNK), jnp.int32),
        pltpu.VMEM((NJH, ECHUNK), jnp.float32),
        pltpu.VMEM((RPT,), jnp.float32),
        pltpu.VMEM_SHARED((NPAD,), jnp.float32),
    ],
)
def _deg_kernel(col_ref, w_ref, deg_ref, col_v, w_v, buf_v, acc_sh):
    cid = lax.axis_index("c")
    sid = lax.axis_index("s")
    pltpu.sync_copy(col_ref.at[sid, pl.ds(cid * NJH, NJH)], col_v)
    pltpu.sync_copy(w_ref.at[sid, pl.ds(cid * NJH, NJH)], w_v)
    ival = jnp.where(cid == 0, 1.0, 0.0).astype(jnp.float32)
    vv = jnp.full((16,), ival, jnp.float32)

    @pl.loop(0, RPT // 16)
    def _(k):
        buf_v[pl.ds(k * 16, 16)] = vv

    pltpu.sync_copy(buf_v, acc_sh.at[pl.ds(sid * RPT, RPT)])
    plsc.subcore_barrier()

    @pl.loop(0, NJH)
    def _(j):
        pltpu.sync_copy(w_v.at[j], acc_sh.at[col_v.at[j]], add=True)

    plsc.subcore_barrier()
    pltpu.sync_copy(acc_sh.at[pl.ds(sid * RPT, RPT)],
                    deg_ref.at[cid, pl.ds(sid * RPT, RPT)])


@functools.partial(
    pl.kernel,
    out_type=jax.ShapeDtypeStruct((2, NPAD, FH), jnp.float32),
    mesh=_mesh,
    scratch_types=[
        pltpu.VMEM((NJ // 2, ECHUNK), jnp.int32),
        pltpu.VMEM((NJ // 2, ECHUNK), jnp.int32),
        pltpu.VMEM((NJ // 2, ECHUNK), jnp.float32),
        pltpu.VMEM((ECHUNK, FH), jnp.float32),
        pltpu.VMEM((ECHUNK, FH), jnp.float32),
        pltpu.VMEM_SHARED((NPAD, FH), jnp.float32),
        pltpu.SemaphoreType.DMA,
        pltpu.SemaphoreType.DMA,
    ],
)
def _prop_kernel(h_ref, row_ref, col_ref, w_ref, agg_ref,
                 row_v, col_v, w_v, gbuf0, gbuf1, acc_sh, sem0, sem1):
    cid = lax.axis_index("c")
    sid = lax.axis_index("s")
    pltpu.sync_copy(h_ref.at[cid, pl.ds(sid * RPT, RPT)],
                    acc_sh.at[pl.ds(sid * RPT, RPT)])
    plsc.subcore_barrier()

    bufs = (gbuf0, gbuf1)
    sems = (sem0, sem1)
    src = h_ref.at[cid]
    NJ2 = NJ // 2

    def _mul_scatter(j, b):
        gbuf = bufs[b]
        pltpu.make_async_copy(src.at[row_v.at[j]], gbuf, sems[b]).wait()
        @pl.when(j + 1 < NJ2)
        def _():
            pltpu.async_copy(src.at[row_v.at[j + 1]],
                             bufs[1 - b], sems[1 - b])

        @pl.loop(0, ECHUNK // 16)
        def _(g):
            wvec = w_v[j, pl.ds(g * 16, 16)]
            for l in range(16):
                wv = jnp.full((16,), wvec[l], jnp.float32)
                e = g * 16 + l
                for k in range(FH // 16):
                    sl = pl.ds(k * 16, 16)
                    gbuf[e, sl] = gbuf[e, sl] * wv

        pltpu.sync_copy(gbuf, acc_sh.at[col_v.at[j]], add=True)

    for half in range(2):
        pltpu.sync_copy(row_ref.at[sid, pl.ds(half * NJ2, NJ2)], row_v)
        pltpu.sync_copy(col_ref.at[sid, pl.ds(half * NJ2, NJ2)], col_v)
        pltpu.sync_copy(w_ref.at[sid, pl.ds(half * NJ2, NJ2)], w_v)
        pltpu.async_copy(src.at[row_v.at[0]], gbuf0, sem0)

        @pl.loop(0, NJ2 // 2)
        def _(h):
            _mul_scatter(2 * h, 0)
            _mul_scatter(2 * h + 1, 1)

    plsc.subcore_barrier()
    pltpu.sync_copy(acc_sh.at[pl.ds(sid * RPT, RPT)],
                    agg_ref.at[cid, pl.ds(sid * RPT, RPT)])


def _dinv_body(deg_ref, o_ref):
    d = deg_ref[0:1, :] + deg_ref[1:2, :]
    o_ref[...] = lax.rsqrt(d)


def _dinv_call(deg2):
    return pl.pallas_call(
        _dinv_body,
        out_shape=jax.ShapeDtypeStruct((1, NPAD), jnp.float32),
    )(deg2)


def _mm1_body(x_ref, w_ref, dinv_ref, o_ref):
    h = jnp.dot(x_ref[...], w_ref[...], precision=lax.Precision.HIGHEST)
    h = h * dinv_ref[...]
    o_ref[0] = h[:, :FH]
    o_ref[1] = h[:, FH:]


def _mm1_call(x_pad, W1, dinv_col):
    return pl.pallas_call(
        _mm1_body,
        grid=(NPAD // 256,),
        in_specs=[
            pl.BlockSpec((256, D), lambda m: (m, 0)),
            pl.BlockSpec((D, D), lambda m: (0, 0)),
            pl.BlockSpec((256, 1), lambda m: (m, 0)),
        ],
        out_specs=pl.BlockSpec((2, 256, FH), lambda m: (0, m, 0)),
        out_shape=jax.ShapeDtypeStruct((2, NPAD, FH), jnp.float32),
    )(x_pad, W1, dinv_col)


def _mid_body(agg_ref, dinv_ref, b1_ref, w2_ref, o_ref):
    u = jnp.concatenate([agg_ref[0], agg_ref[1]], axis=1)
    out1 = jax.nn.relu(u * dinv_ref[...] + b1_ref[...])
    h2 = jnp.dot(out1, w2_ref[...], precision=lax.Precision.HIGHEST)
    h2 = h2 * dinv_ref[...]
    o_ref[0] = h2[:, :FH]
    o_ref[1] = h2[:, FH:]


def _mid_call(agg1, dinv_col, b1, W2):
    return pl.pallas_call(
        _mid_body,
        grid=(NPAD // 256,),
        in_specs=[
            pl.BlockSpec((2, 256, FH), lambda m: (0, m, 0)),
            pl.BlockSpec((256, 1), lambda m: (m, 0)),
            pl.BlockSpec((1, D), lambda m: (0, 0)),
            pl.BlockSpec((D, D), lambda m: (0, 0)),
        ],
        out_specs=pl.BlockSpec((2, 256, FH), lambda m: (0, m, 0)),
        out_shape=jax.ShapeDtypeStruct((2, NPAD, FH), jnp.float32),
    )(agg1, dinv_col, b1, W2)


def _fin_body(agg_ref, dinv_ref, b2_ref, o_ref):
    u = jnp.concatenate([agg_ref[0], agg_ref[1]], axis=1)
    o_ref[...] = u * dinv_ref[...] + b2_ref[...]


def _fin_call(agg2, dinv_col, b2):
    return pl.pallas_call(
        _fin_body,
        grid=(NPAD // 256,),
        in_specs=[
            pl.BlockSpec((2, 256, FH), lambda m: (0, m, 0)),
            pl.BlockSpec((256, 1), lambda m: (m, 0)),
            pl.BlockSpec((1, D), lambda m: (0, 0)),
        ],
        out_specs=pl.BlockSpec((256, D), lambda m: (m, 0)),
        out_shape=jax.ShapeDtypeStruct((NPAD, D), jnp.float32),
    )(agg2, dinv_col, b2)


def kernel(x, edge_index, edge_weight, W1, b1, W2, b2):
    row = edge_index[0].astype(jnp.int32)
    col = edge_index[1].astype(jnp.int32)
    w = edge_weight.astype(jnp.float32)
    pad = EPAD - E
    padidx = N + (jnp.arange(pad, dtype=jnp.int32) % (NPAD - N))
    row_t = jnp.concatenate([row, padidx]).reshape(NSLAB, NJ, ECHUNK)
    col_t = jnp.concatenate([col, padidx]).reshape(NSLAB, NJ, ECHUNK)
    w_t = jnp.concatenate([w, jnp.zeros((pad,), jnp.float32)]
                          ).reshape(NSLAB, NJ, ECHUNK)
    x_pad = jnp.pad(x, ((0, NPAD - N), (0, 0)))
    b1r = b1.reshape(1, D)
    b2r = b2.reshape(1, D)

    deg2 = _deg_kernel(col_t, w_t)
    dinv_col = _dinv_call(deg2).reshape(NPAD, 1)
    h1 = _mm1_call(x_pad, W1, dinv_col)
    agg1 = _prop_kernel(h1, row_t, col_t, w_t)
    h2 = _mid_call(agg1, dinv_col, b1r, W2)
    agg2 = _prop_kernel(h2, row_t, col_t, w_t)
    out = _fin_call(agg2, dinv_col, b2r)
    return out[:N]


# speedup vs baseline: 1.0867x; 1.0286x over previous
import functools

import jax
import jax.numpy as jnp
from jax import lax
from jax.experimental import pallas as pl
from jax.experimental.pallas import tpu as pltpu
from jax.experimental.pallas import tpu_sc as plsc

N = 10000
E = 160000
D = 256

NPAD = 10240
RPT = NPAD // 16
ECHUNK = 128
NSLAB = 16
NJ = 80
EPAD = NSLAB * NJ * ECHUNK
NJH = NJ // 2
FH = 128

_mesh = plsc.VectorSubcoreMesh(core_axis_name="c", subcore_axis_name="s")


@functools.partial(
    pl.kernel,
    out_type=jax.ShapeDtypeStruct((2, NPAD), jnp.float32),
    mesh=_mesh,
    scratch_types=[
        pltpu.VMEM((NJH, ECHUNK), jnp.int32),
        pltpu.VMEM((NJH, ECHUNK), jnp.float32),
        pltpu.VMEM((RPT,), jnp.float32),
        pltpu.VMEM_SHARED((NPAD,), jnp.float32),
    ],
)
def _deg_kernel(col_ref, w_ref, deg_ref, col_v, w_v, buf_v, acc_sh):
    cid = lax.axis_index("c")
    sid = lax.axis_index("s")
    pltpu.sync_copy(col_ref.at[sid, pl.ds(cid * NJH, NJH)], col_v)
    pltpu.sync_copy(w_ref.at[sid, pl.ds(cid * NJH, NJH)], w_v)
    ival = jnp.where(cid == 0, 1.0, 0.0).astype(jnp.float32)
    vv = jnp.full((16,), ival, jnp.float32)

    @pl.loop(0, RPT // 16)
    def _(k):
        buf_v[pl.ds(k * 16, 16)] = vv

    pltpu.sync_copy(buf_v, acc_sh.at[pl.ds(sid * RPT, RPT)])
    plsc.subcore_barrier()

    @pl.loop(0, NJH)
    def _(j):
        pltpu.sync_copy(w_v.at[j], acc_sh.at[col_v.at[j]], add=True)

    plsc.subcore_barrier()
    pltpu.sync_copy(acc_sh.at[pl.ds(sid * RPT, RPT)],
                    deg_ref.at[cid, pl.ds(sid * RPT, RPT)])


@functools.partial(
    pl.kernel,
    out_type=jax.ShapeDtypeStruct((2, NPAD, FH), jnp.float32),
    mesh=_mesh,
    scratch_types=[
        pltpu.VMEM((NJ // 2, ECHUNK), jnp.int32),
        pltpu.VMEM((NJ // 2, ECHUNK), jnp.int32),
        pltpu.VMEM((NJ // 2, ECHUNK), jnp.float32),
        pltpu.VMEM((ECHUNK, FH), jnp.float32),
        pltpu.VMEM((ECHUNK, FH), jnp.float32),
        pltpu.VMEM_SHARED((NPAD, FH), jnp.float32),
        pltpu.SemaphoreType.DMA,
        pltpu.SemaphoreType.DMA,
    ],
)
def _prop_kernel(h_ref, row_ref, col_ref, w_ref, agg_ref,
                 row_v, col_v, w_v, gbuf0, gbuf1, acc_sh, sem0, sem1):
    cid = lax.axis_index("c")
    sid = lax.axis_index("s")
    pltpu.sync_copy(h_ref.at[cid, pl.ds(sid * RPT, RPT)],
                    acc_sh.at[pl.ds(sid * RPT, RPT)])
    plsc.subcore_barrier()

    bufs = (gbuf0, gbuf1)
    sems = (sem0, sem1)
    src = h_ref.at[cid]
    NJ2 = NJ // 2

    def _mul_scatter(j, b):
        gbuf = bufs[b]
        pltpu.make_async_copy(src.at[row_v.at[j]], gbuf, sems[b]).wait()
        @pl.when(j + 1 < NJ2)
        def _():
            pltpu.async_copy(src.at[row_v.at[j + 1]],
                             bufs[1 - b], sems[1 - b])

        @pl.loop(0, ECHUNK // 16)
        def _(g):
            wvec = w_v[j, pl.ds(g * 16, 16)]
            for l in range(16):
                wv = jnp.full((16,), wvec[l], jnp.float32)
                e = g * 16 + l
                for k in range(FH // 16):
                    sl = pl.ds(k * 16, 16)
                    gbuf[e, sl] = gbuf[e, sl] * wv

        pltpu.sync_copy(gbuf, acc_sh.at[col_v.at[j]], add=True)

    for half in range(2):
        pltpu.sync_copy(row_ref.at[sid, pl.ds(half * NJ2, NJ2)], row_v)
        pltpu.sync_copy(col_ref.at[sid, pl.ds(half * NJ2, NJ2)], col_v)
        pltpu.sync_copy(w_ref.at[sid, pl.ds(half * NJ2, NJ2)], w_v)
        pltpu.async_copy(src.at[row_v.at[0]], gbuf0, sem0)

        @pl.loop(0, NJ2 // 2)
        def _(h):
            _mul_scatter(2 * h, 0)
            _mul_scatter(2 * h + 1, 1)

    plsc.subcore_barrier()
    pltpu.sync_copy(acc_sh.at[pl.ds(sid * RPT, RPT)],
                    agg_ref.at[cid, pl.ds(sid * RPT, RPT)])


def _dinv_body(deg_ref, o_ref):
    d = deg_ref[0:1, :] + deg_ref[1:2, :]
    o_ref[...] = lax.rsqrt(d)


def _dinv_call(deg2):
    return pl.pallas_call(
        _dinv_body,
        out_shape=jax.ShapeDtypeStruct((1, NPAD), jnp.float32),
    )(deg2)


def _mm1_body(x_ref, w_ref, dinv_ref, o_ref):
    h = jnp.dot(x_ref[...], w_ref[...])
    h = h * dinv_ref[...]
    o_ref[0] = h[:, :FH]
    o_ref[1] = h[:, FH:]


def _mm1_call(x_pad, W1, dinv_col):
    return pl.pallas_call(
        _mm1_body,
        grid=(NPAD // 256,),
        in_specs=[
            pl.BlockSpec((256, D), lambda m: (m, 0)),
            pl.BlockSpec((D, D), lambda m: (0, 0)),
            pl.BlockSpec((256, 1), lambda m: (m, 0)),
        ],
        out_specs=pl.BlockSpec((2, 256, FH), lambda m: (0, m, 0)),
        out_shape=jax.ShapeDtypeStruct((2, NPAD, FH), jnp.float32),
    )(x_pad, W1, dinv_col)


def _mid_body(agg_ref, dinv_ref, b1_ref, w2_ref, o_ref):
    u = jnp.concatenate([agg_ref[0], agg_ref[1]], axis=1)
    out1 = jax.nn.relu(u * dinv_ref[...] + b1_ref[...])
    h2 = jnp.dot(out1, w2_ref[...])
    h2 = h2 * dinv_ref[...]
    o_ref[0] = h2[:, :FH]
    o_ref[1] = h2[:, FH:]


def _mid_call(agg1, dinv_col, b1, W2):
    return pl.pallas_call(
        _mid_body,
        grid=(NPAD // 256,),
        in_specs=[
            pl.BlockSpec((2, 256, FH), lambda m: (0, m, 0)),
            pl.BlockSpec((256, 1), lambda m: (m, 0)),
            pl.BlockSpec((1, D), lambda m: (0, 0)),
            pl.BlockSpec((D, D), lambda m: (0, 0)),
        ],
        out_specs=pl.BlockSpec((2, 256, FH), lambda m: (0, m, 0)),
        out_shape=jax.ShapeDtypeStruct((2, NPAD, FH), jnp.float32),
    )(agg1, dinv_col, b1, W2)


def _fin_body(agg_ref, dinv_ref, b2_ref, o_ref):
    u = jnp.concatenate([agg_ref[0], agg_ref[1]], axis=1)
    o_ref[...] = u * dinv_ref[...] + b2_ref[...]


def _fin_call(agg2, dinv_col, b2):
    return pl.pallas_call(
        _fin_body,
        grid=(NPAD // 256,),
        in_specs=[
            pl.BlockSpec((2, 256, FH), lambda m: (0, m, 0)),
            pl.BlockSpec((256, 1), lambda m: (m, 0)),
            pl.BlockSpec((1, D), lambda m: (0, 0)),
        ],
        out_specs=pl.BlockSpec((256, D), lambda m: (m, 0)),
        out_shape=jax.ShapeDtypeStruct((NPAD, D), jnp.float32),
    )(agg2, dinv_col, b2)


def kernel(x, edge_index, edge_weight, W1, b1, W2, b2):
    row = edge_index[0].astype(jnp.int32)
    col = edge_index[1].astype(jnp.int32)
    w = edge_weight.astype(jnp.float32)
    pad = EPAD - E
    padidx = N + (jnp.arange(pad, dtype=jnp.int32) % (NPAD - N))
    row_t = jnp.concatenate([row, padidx]).reshape(NSLAB, NJ, ECHUNK)
    col_t = jnp.concatenate([col, padidx]).reshape(NSLAB, NJ, ECHUNK)
    w_t = jnp.concatenate([w, jnp.zeros((pad,), jnp.float32)]
                          ).reshape(NSLAB, NJ, ECHUNK)
    x_pad = jnp.pad(x, ((0, NPAD - N), (0, 0)))
    b1r = b1.reshape(1, D)
    b2r = b2.reshape(1, D)

    deg2 = _deg_kernel(col_t, w_t)
    dinv_col = _dinv_call(deg2).reshape(NPAD, 1)
    h1 = _mm1_call(x_pad, W1, dinv_col)
    agg1 = _prop_kernel(h1, row_t, col_t, w_t)
    h2 = _mid_call(agg1, dinv_col, b1r, W2)
    agg2 = _prop_kernel(h2, row_t, col_t, w_t)
    out = _fin_call(agg2, dinv_col, b2r)
    return out[:N]
